# baseline (speedup 1.0000x reference)
import functools

import jax
import jax.numpy as jnp
from jax import lax
from jax.experimental import pallas as pl
from jax.experimental.pallas import tpu as pltpu
from jax.experimental.pallas import tpu_sc as plsc

N_NODES = 10000
D = 128
E = 320000
W_EDGES = 125
ROWS = E // W_EDGES
NC, NS = 2, 16
ROWS_PER_W = ROWS // (NC * NS)
NODE_BASE = 624
CH = 16
IB = 8


def _sc_mesh():
    return plsc.VectorSubcoreMesh(core_axis_name="c", subcore_axis_name="s",
                                  num_cores=NC, num_subcores=NS)


def _fill(ref, rows, value):
    @pl.loop(0, rows)
    def _(r):
        @pl.loop(0, D, step=16)
        def _(k):
            ref[r, pl.ds(k, 16)] = jnp.full((16,), value, jnp.float32)


def _own_range(s):
    base_n = s * NODE_BASE
    n_own = jnp.where(s == NS - 1, N_NODES - (NS - 1) * NODE_BASE, NODE_BASE)
    return base_n, n_own


def _zero_acc(acc_sh, zsrc, base_n, n_own):
    @pl.loop(0, n_own, step=CH)
    def _(r):
        pltpu.sync_copy(zsrc, acc_sh.at[pl.ds(base_n + r, CH)])


def _copy_out(acc_sh, out_hbm, c, base_n, n_own):
    @pl.loop(0, n_own, step=CH)
    def _(r):
        pltpu.sync_copy(acc_sh.at[pl.ds(base_n + r, CH)],
                        out_hbm.at[c].at[pl.ds(base_n + r, CH)])


def _make_agg():
    def body(feat_hbm, src_hbm, dst_hbm, out_hbm, src_v, dst_v,
             rows_a, rows_b, acc_sh, sem_ga, sem_gb, sem_sa, sem_sb):
        c = lax.axis_index("c")
        s = lax.axis_index("s")
        base_n, n_own = _own_range(s)

        _fill(rows_a, W_EDGES, 0.0)
        _zero_acc(acc_sh, rows_a.at[pl.ds(0, CH)], base_n, n_own)
        plsc.subcore_barrier()

        row_base = (c * NS + s) * ROWS_PER_W
        bufs = (rows_a, rows_b)
        gsems = (sem_ga, sem_gb)
        ssems = (sem_sa, sem_sb)

        @pl.loop(0, ROWS_PER_W, step=IB)
        def _(b):
            pltpu.sync_copy(src_hbm.at[pl.ds(row_base + b, IB)], src_v)
            pltpu.sync_copy(dst_hbm.at[pl.ds(row_base + b, IB)], dst_v)

            gd = [None] * IB
            sd = [None] * IB
            gd[0] = pltpu.async_copy(
                feat_hbm.at[src_v.at[0]], bufs[0], gsems[0])
            for j in range(IB):
                gd[j].wait()
                sd[j] = pltpu.async_copy(
                    bufs[j % 2], acc_sh.at[dst_v.at[j]], ssems[j % 2],
                    add=True)
                if j + 1 < IB:
                    if j >= 1:
                        sd[j - 1].wait()
                    gd[j + 1] = pltpu.async_copy(
                        feat_hbm.at[src_v.at[j + 1]],
                        bufs[(j + 1) % 2], gsems[(j + 1) % 2])
            sd[IB - 2].wait()
            sd[IB - 1].wait()

        plsc.subcore_barrier()
        _copy_out(acc_sh, out_hbm, c, base_n, n_own)

    return pl.kernel(
        body,
        out_type=jax.ShapeDtypeStruct((NC, N_NODES, D), jnp.float32),
        mesh=_sc_mesh(),
        scratch_types=[
            pltpu.VMEM((IB, W_EDGES), jnp.int32),
            pltpu.VMEM((IB, W_EDGES), jnp.int32),
            pltpu.VMEM((W_EDGES, D), jnp.float32),
            pltpu.VMEM((W_EDGES, D), jnp.float32),
            pltpu.VMEM_SHARED((N_NODES, D), jnp.float32),
            pltpu.SemaphoreType.DMA,
            pltpu.SemaphoreType.DMA,
            pltpu.SemaphoreType.DMA,
            pltpu.SemaphoreType.DMA,
        ])


def _make_deg():
    def body(dst_hbm, out_hbm, dst_v, ones_v, zbuf, acc_sh, dsem):
        c = lax.axis_index("c")
        s = lax.axis_index("s")
        base_n, n_own = _own_range(s)

        _fill(zbuf, CH, 0.0)
        _fill(ones_v, W_EDGES, 1.0)
        _zero_acc(acc_sh, zbuf, base_n, n_own)
        plsc.subcore_barrier()

        row_base = (c * NS + s) * ROWS_PER_W

        @pl.loop(0, ROWS_PER_W, step=IB)
        def _(b):
            pltpu.sync_copy(dst_hbm.at[pl.ds(row_base + b, IB)], dst_v)

            descs = [pltpu.async_copy(ones_v, acc_sh.at[dst_v.at[j]],
                                      dsem, add=True)
                     for j in range(IB)]
            for d in descs:
                d.wait()

        plsc.subcore_barrier()
        _copy_out(acc_sh, out_hbm, c, base_n, n_own)

    return pl.kernel(
        body,
        out_type=jax.ShapeDtypeStruct((NC, N_NODES, D), jnp.float32),
        mesh=_sc_mesh(),
        scratch_types=[
            pltpu.VMEM((IB, W_EDGES), jnp.int32),
            pltpu.VMEM((W_EDGES, D), jnp.float32),
            pltpu.VMEM((CH, D), jnp.float32),
            pltpu.VMEM_SHARED((N_NODES, D), jnp.float32),
            pltpu.SemaphoreType.DMA,
        ])


_agg = functools.cache(_make_agg)
_deg = functools.cache(_make_deg)

BLK = 2000


def _mlp_block(x_ref, p_ref, dg_ref, wa_ref, ba_ref, wb_ref, bb_ref, o_ref):
    deg = dg_ref[0, :, 0:1] + dg_ref[1, :, 0:1]
    inv = 1.0 / jnp.maximum(deg, 1.0)
    h = x_ref[...] + (p_ref[0] + p_ref[1]) * inv
    t = jnp.dot(h, wa_ref[...], preferred_element_type=jnp.float32) + ba_ref[...]
    t = jnp.dot(t, wb_ref[...], preferred_element_type=jnp.float32) + bb_ref[...]
    o_ref[...] = jnp.maximum(t, 0.0)


def _mlp(x, p, degp, wa_t, ba, wb_t, bb):
    return pl.pallas_call(
        _mlp_block,
        grid=(N_NODES // BLK,),
        in_specs=[
            pl.BlockSpec((BLK, D), lambda i: (i, 0)),
            pl.BlockSpec((NC, BLK, D), lambda i: (0, i, 0)),
            pl.BlockSpec((NC, BLK, D), lambda i: (0, i, 0)),
            pl.BlockSpec((D, D), lambda i: (0, 0)),
            pl.BlockSpec((1, D), lambda i: (0, 0)),
            pl.BlockSpec((D, D), lambda i: (0, 0)),
            pl.BlockSpec((1, D), lambda i: (0, 0)),
        ],
        out_specs=pl.BlockSpec((BLK, D), lambda i: (i, 0)),
        out_shape=jax.ShapeDtypeStruct((N_NODES, D), jnp.float32),
    )(x, p, degp, wa_t, ba.reshape(1, D), wb_t, bb.reshape(1, D))


def kernel(features, edge_index, W1a, b1a, W1b, b1b, W2a, b2a, W2b, b2b):
    src = edge_index[0].astype(jnp.int32).reshape(ROWS, W_EDGES)
    dst = edge_index[1].astype(jnp.int32).reshape(ROWS, W_EDGES)
    degp = _deg()(dst)
    p1 = _agg()(features, src, dst)
    x1 = _mlp(features, p1, degp, W1a.T, b1a, W1b.T, b1b)
    p2 = _agg()(x1, src, dst)
    out = _mlp(x1, p2, degp, W2a.T, b2a, W2b.T, b2b)
    return out


# speedup vs baseline: 1.1289x; 1.1289x over previous
import functools

import jax
import jax.numpy as jnp
from jax import lax
from jax.experimental import pallas as pl
from jax.experimental.pallas import tpu as pltpu
from jax.experimental.pallas import tpu_sc as plsc

N_NODES = 10000
D = 128
E = 320000
W_EDGES = 125
ROWS = E // W_EDGES
NC, NS = 2, 16
ROWS_PER_W = ROWS // (NC * NS)
NODE_BASE = 624
CH = 16
IB = 16


def _sc_mesh():
    return plsc.VectorSubcoreMesh(core_axis_name="c", subcore_axis_name="s",
                                  num_cores=NC, num_subcores=NS)


def _fill(ref, rows, value):
    @pl.loop(0, rows)
    def _(r):
        @pl.loop(0, D, step=16)
        def _(k):
            ref[r, pl.ds(k, 16)] = jnp.full((16,), value, jnp.float32)


def _own_range(s):
    base_n = s * NODE_BASE
    n_own = jnp.where(s == NS - 1, N_NODES - (NS - 1) * NODE_BASE, NODE_BASE)
    return base_n, n_own


def _zero_acc(acc_sh, zsrc, base_n, n_own):
    @pl.loop(0, n_own, step=CH)
    def _(r):
        pltpu.sync_copy(zsrc, acc_sh.at[pl.ds(base_n + r, CH)])


def _copy_out(acc_sh, out_hbm, c, base_n, n_own):
    @pl.loop(0, n_own, step=CH)
    def _(r):
        pltpu.sync_copy(acc_sh.at[pl.ds(base_n + r, CH)],
                        out_hbm.at[c].at[pl.ds(base_n + r, CH)])


def _make_agg():
    def body(feat_hbm, src_hbm, dst_hbm, out_hbm, src_v, dst_v,
             rows_a, rows_b, acc_sh, sem_ga, sem_gb, sem_sa, sem_sb):
        c = lax.axis_index("c")
        s = lax.axis_index("s")
        base_n, n_own = _own_range(s)

        _fill(rows_a, W_EDGES, 0.0)
        _zero_acc(acc_sh, rows_a.at[pl.ds(0, CH)], base_n, n_own)
        plsc.subcore_barrier()

        row_base = (c * NS + s) * ROWS_PER_W
        bufs = (rows_a, rows_b)
        gsems = (sem_ga, sem_gb)
        ssems = (sem_sa, sem_sb)

        @pl.loop(0, ROWS_PER_W, step=IB)
        def _(b):
            pltpu.sync_copy(src_hbm.at[pl.ds(row_base + b, IB)], src_v)
            pltpu.sync_copy(dst_hbm.at[pl.ds(row_base + b, IB)], dst_v)

            gd = [None] * IB
            gd[0] = pltpu.async_copy(
                feat_hbm.at[src_v.at[0]], bufs[0], gsems[0])
            for j in range(IB):
                if j + 1 < IB:
                    gd[j + 1] = pltpu.async_copy(
                        feat_hbm.at[src_v.at[j + 1]],
                        bufs[(j + 1) % 2], gsems[(j + 1) % 2])
                gd[j].wait()
                pltpu.sync_copy(bufs[j % 2], acc_sh.at[dst_v.at[j]], add=True)

        plsc.subcore_barrier()
        _copy_out(acc_sh, out_hbm, c, base_n, n_own)

    return pl.kernel(
        body,
        out_type=jax.ShapeDtypeStruct((NC, N_NODES, D), jnp.float32),
        mesh=_sc_mesh(),
        scratch_types=[
            pltpu.VMEM((IB, W_EDGES), jnp.int32),
            pltpu.VMEM((IB, W_EDGES), jnp.int32),
            pltpu.VMEM((W_EDGES, D), jnp.float32),
            pltpu.VMEM((W_EDGES, D), jnp.float32),
            pltpu.VMEM_SHARED((N_NODES, D), jnp.float32),
            pltpu.SemaphoreType.DMA,
            pltpu.SemaphoreType.DMA,
            pltpu.SemaphoreType.DMA,
            pltpu.SemaphoreType.DMA,
        ])


def _make_deg():
    def body(dst_hbm, out_hbm, dst_v, ones_v, zbuf, acc_sh, dsem):
        c = lax.axis_index("c")
        s = lax.axis_index("s")
        base_n, n_own = _own_range(s)

        _fill(zbuf, CH, 0.0)
        _fill(ones_v, W_EDGES, 1.0)
        _zero_acc(acc_sh, zbuf, base_n, n_own)
        plsc.subcore_barrier()

        row_base = (c * NS + s) * ROWS_PER_W

        @pl.loop(0, ROWS_PER_W, step=IB)
        def _(b):
            pltpu.sync_copy(dst_hbm.at[pl.ds(row_base + b, IB)], dst_v)

            @pl.loop(0, IB)
            def _(j):
                pltpu.sync_copy(ones_v, acc_sh.at[dst_v.at[j]], add=True)

        plsc.subcore_barrier()
        _copy_out(acc_sh, out_hbm, c, base_n, n_own)

    return pl.kernel(
        body,
        out_type=jax.ShapeDtypeStruct((NC, N_NODES, D), jnp.float32),
        mesh=_sc_mesh(),
        scratch_types=[
            pltpu.VMEM((IB, W_EDGES), jnp.int32),
            pltpu.VMEM((W_EDGES, D), jnp.float32),
            pltpu.VMEM((CH, D), jnp.float32),
            pltpu.VMEM_SHARED((N_NODES, D), jnp.float32),
            pltpu.SemaphoreType.DMA,
        ])


_agg = functools.cache(_make_agg)
_deg = functools.cache(_make_deg)

BLK = 2000


def _mlp_block(x_ref, p_ref, dg_ref, wa_ref, ba_ref, wb_ref, bb_ref, o_ref):
    deg = dg_ref[0, :, 0:1] + dg_ref[1, :, 0:1]
    inv = 1.0 / jnp.maximum(deg, 1.0)
    h = x_ref[...] + (p_ref[0] + p_ref[1]) * inv
    t = jnp.dot(h, wa_ref[...], preferred_element_type=jnp.float32) + ba_ref[...]
    t = jnp.dot(t, wb_ref[...], preferred_element_type=jnp.float32) + bb_ref[...]
    o_ref[...] = jnp.maximum(t, 0.0)


def _mlp(x, p, degp, wa_t, ba, wb_t, bb):
    return pl.pallas_call(
        _mlp_block,
        grid=(N_NODES // BLK,),
        in_specs=[
            pl.BlockSpec((BLK, D), lambda i: (i, 0)),
            pl.BlockSpec((NC, BLK, D), lambda i: (0, i, 0)),
            pl.BlockSpec((NC, BLK, D), lambda i: (0, i, 0)),
            pl.BlockSpec((D, D), lambda i: (0, 0)),
            pl.BlockSpec((1, D), lambda i: (0, 0)),
            pl.BlockSpec((D, D), lambda i: (0, 0)),
            pl.BlockSpec((1, D), lambda i: (0, 0)),
        ],
        out_specs=pl.BlockSpec((BLK, D), lambda i: (i, 0)),
        out_shape=jax.ShapeDtypeStruct((N_NODES, D), jnp.float32),
    )(x, p, degp, wa_t, ba.reshape(1, D), wb_t, bb.reshape(1, D))


def kernel(features, edge_index, W1a, b1a, W1b, b1b, W2a, b2a, W2b, b2b):
    src = edge_index[0].astype(jnp.int32).reshape(ROWS, W_EDGES)
    dst = edge_index[1].astype(jnp.int32).reshape(ROWS, W_EDGES)
    degp = _deg()(dst)
    p1 = _agg()(features, src, dst)
    x1 = _mlp(features, p1, degp, W1a.T, b1a, W1b.T, b1b)
    p2 = _agg()(x1, src, dst)
    out = _mlp(x1, p2, degp, W2a.T, b2a, W2b.T, b2b)
    return out


# speedup vs baseline: 1.1792x; 1.0446x over previous
import functools

import jax
import jax.numpy as jnp
from jax import lax
from jax.experimental import pallas as pl
from jax.experimental.pallas import tpu as pltpu
from jax.experimental.pallas import tpu_sc as plsc

N_NODES = 10000
D = 128
E = 320000
W_EDGES = 125
ROWS = E // W_EDGES
NC, NS = 2, 16
ROWS_PER_W = ROWS // (NC * NS)
NODE_BASE = 624
CH = 16
IB = 40


def _sc_mesh():
    return plsc.VectorSubcoreMesh(core_axis_name="c", subcore_axis_name="s",
                                  num_cores=NC, num_subcores=NS)


def _fill(ref, rows, value):
    @pl.loop(0, rows)
    def _(r):
        @pl.loop(0, D, step=16)
        def _(k):
            ref[r, pl.ds(k, 16)] = jnp.full((16,), value, jnp.float32)


def _own_range(s):
    base_n = s * NODE_BASE
    n_own = jnp.where(s == NS - 1, N_NODES - (NS - 1) * NODE_BASE, NODE_BASE)
    return base_n, n_own


def _zero_acc(acc_sh, zsrc, base_n, n_own):
    @pl.loop(0, n_own, step=CH)
    def _(r):
        pltpu.sync_copy(zsrc, acc_sh.at[pl.ds(base_n + r, CH)])


def _copy_out(acc_sh, out_hbm, c, base_n, n_own):
    @pl.loop(0, n_own, step=CH)
    def _(r):
        pltpu.sync_copy(acc_sh.at[pl.ds(base_n + r, CH)],
                        out_hbm.at[c].at[pl.ds(base_n + r, CH)])


def _make_agg():
    def body(feat_hbm, src_hbm, dst_hbm, out_hbm, src_v, dst_v,
             rows_a, rows_b, acc_sh, sem_ga, sem_gb, sem_sa, sem_sb):
        c = lax.axis_index("c")
        s = lax.axis_index("s")
        base_n, n_own = _own_range(s)

        _fill(rows_a, W_EDGES, 0.0)
        _zero_acc(acc_sh, rows_a.at[pl.ds(0, CH)], base_n, n_own)
        plsc.subcore_barrier()

        row_base = (c * NS + s) * ROWS_PER_W
        bufs = (rows_a, rows_b)
        gsems = (sem_ga, sem_gb)
        ssems = (sem_sa, sem_sb)

        @pl.loop(0, ROWS_PER_W, step=IB)
        def _(b):
            pltpu.sync_copy(src_hbm.at[pl.ds(row_base + b, IB)], src_v)
            pltpu.sync_copy(dst_hbm.at[pl.ds(row_base + b, IB)], dst_v)

            gd = [None] * IB
            gd[0] = pltpu.async_copy(
                feat_hbm.at[src_v.at[0]], bufs[0], gsems[0])
            for j in range(IB):
                if j + 1 < IB:
                    gd[j + 1] = pltpu.async_copy(
                        feat_hbm.at[src_v.at[j + 1]],
                        bufs[(j + 1) % 2], gsems[(j + 1) % 2])
                gd[j].wait()
                pltpu.sync_copy(bufs[j % 2], acc_sh.at[dst_v.at[j]], add=True)

        plsc.subcore_barrier()
        _copy_out(acc_sh, out_hbm, c, base_n, n_own)

    return pl.kernel(
        body,
        out_type=jax.ShapeDtypeStruct((NC, N_NODES, D), jnp.float32),
        mesh=_sc_mesh(),
        scratch_types=[
            pltpu.VMEM((IB, W_EDGES), jnp.int32),
            pltpu.VMEM((IB, W_EDGES), jnp.int32),
            pltpu.VMEM((W_EDGES, D), jnp.float32),
            pltpu.VMEM((W_EDGES, D), jnp.float32),
            pltpu.VMEM_SHARED((N_NODES, D), jnp.float32),
            pltpu.SemaphoreType.DMA,
            pltpu.SemaphoreType.DMA,
            pltpu.SemaphoreType.DMA,
            pltpu.SemaphoreType.DMA,
        ])


def _make_deg():
    def body(dst_hbm, out_hbm, dst_v, ones_v, zbuf, acc_sh, dsem):
        c = lax.axis_index("c")
        s = lax.axis_index("s")
        base_n, n_own = _own_range(s)

        _fill(zbuf, CH, 0.0)
        _fill(ones_v, W_EDGES, 1.0)
        _zero_acc(acc_sh, zbuf, base_n, n_own)
        plsc.subcore_barrier()

        row_base = (c * NS + s) * ROWS_PER_W

        @pl.loop(0, ROWS_PER_W, step=IB)
        def _(b):
            pltpu.sync_copy(dst_hbm.at[pl.ds(row_base + b, IB)], dst_v)

            @pl.loop(0, IB)
            def _(j):
                pltpu.sync_copy(ones_v, acc_sh.at[dst_v.at[j]], add=True)

        plsc.subcore_barrier()
        _copy_out(acc_sh, out_hbm, c, base_n, n_own)

    return pl.kernel(
        body,
        out_type=jax.ShapeDtypeStruct((NC, N_NODES, D), jnp.float32),
        mesh=_sc_mesh(),
        scratch_types=[
            pltpu.VMEM((IB, W_EDGES), jnp.int32),
            pltpu.VMEM((W_EDGES, D), jnp.float32),
            pltpu.VMEM((CH, D), jnp.float32),
            pltpu.VMEM_SHARED((N_NODES, D), jnp.float32),
            pltpu.SemaphoreType.DMA,
        ])


_agg = functools.cache(_make_agg)
_deg = functools.cache(_make_deg)

BLK = 2000


def _mlp_block(x_ref, p_ref, dg_ref, wa_ref, ba_ref, wb_ref, bb_ref, o_ref):
    deg = dg_ref[0, :, 0:1] + dg_ref[1, :, 0:1]
    inv = 1.0 / jnp.maximum(deg, 1.0)
    h = x_ref[...] + (p_ref[0] + p_ref[1]) * inv
    t = jnp.dot(h, wa_ref[...], preferred_element_type=jnp.float32) + ba_ref[...]
    t = jnp.dot(t, wb_ref[...], preferred_element_type=jnp.float32) + bb_ref[...]
    o_ref[...] = jnp.maximum(t, 0.0)


def _mlp(x, p, degp, wa_t, ba, wb_t, bb):
    return pl.pallas_call(
        _mlp_block,
        grid=(N_NODES // BLK,),
        in_specs=[
            pl.BlockSpec((BLK, D), lambda i: (i, 0)),
            pl.BlockSpec((NC, BLK, D), lambda i: (0, i, 0)),
            pl.BlockSpec((NC, BLK, D), lambda i: (0, i, 0)),
            pl.BlockSpec((D, D), lambda i: (0, 0)),
            pl.BlockSpec((1, D), lambda i: (0, 0)),
            pl.BlockSpec((D, D), lambda i: (0, 0)),
            pl.BlockSpec((1, D), lambda i: (0, 0)),
        ],
        out_specs=pl.BlockSpec((BLK, D), lambda i: (i, 0)),
        out_shape=jax.ShapeDtypeStruct((N_NODES, D), jnp.float32),
    )(x, p, degp, wa_t, ba.reshape(1, D), wb_t, bb.reshape(1, D))


def kernel(features, edge_index, W1a, b1a, W1b, b1b, W2a, b2a, W2b, b2b):
    src = edge_index[0].astype(jnp.int32).reshape(ROWS, W_EDGES)
    dst = edge_index[1].astype(jnp.int32).reshape(ROWS, W_EDGES)
    degp = _deg()(dst)
    p1 = _agg()(features, src, dst)
    x1 = _mlp(features, p1, degp, W1a.T, b1a, W1b.T, b1b)
    p2 = _agg()(x1, src, dst)
    out = _mlp(x1, p2, degp, W2a.T, b2a, W2b.T, b2b)
    return out


# speedup vs baseline: 1.3857x; 1.1751x over previous
import functools

import jax
import jax.numpy as jnp
from jax import lax
from jax.experimental import pallas as pl
from jax.experimental.pallas import tpu as pltpu
from jax.experimental.pallas import tpu_sc as plsc

N_NODES = 10000
D = 128
E = 320000
W_EDGES = 125
ROWS = E // W_EDGES
NC, NS = 2, 16
ROWS_PER_W = ROWS // (NC * NS)
NODE_BASE = 624
CH = 16
IB = 40


def _sc_mesh():
    return plsc.VectorSubcoreMesh(core_axis_name="c", subcore_axis_name="s",
                                  num_cores=NC, num_subcores=NS)


def _fill(ref, rows, value):
    @pl.loop(0, rows)
    def _(r):
        @pl.loop(0, D, step=16)
        def _(k):
            ref[r, pl.ds(k, 16)] = jnp.full((16,), value, jnp.float32)


def _own_range(s):
    base_n = s * NODE_BASE
    n_own = jnp.where(s == NS - 1, N_NODES - (NS - 1) * NODE_BASE, NODE_BASE)
    return base_n, n_own


CHZ = 104
CHO = 208


def _zero_acc(acc_sh, zsrc_big, zsrc_rem, base_n, s):
    @pl.loop(0, NODE_BASE, step=CHZ)
    def _(r):
        pltpu.sync_copy(zsrc_big, acc_sh.at[pl.ds(base_n + r, CHZ)])

    @pl.when(s == NS - 1)
    def _():
        pltpu.sync_copy(zsrc_rem, acc_sh.at[pl.ds(base_n + NODE_BASE, CH)])


def _copy_out(acc_sh, out_hbm, c, base_n, s):
    @pl.loop(0, NODE_BASE, step=CHO)
    def _(r):
        pltpu.sync_copy(acc_sh.at[pl.ds(base_n + r, CHO)],
                        out_hbm.at[c].at[pl.ds(base_n + r, CHO)])

    @pl.when(s == NS - 1)
    def _():
        pltpu.sync_copy(acc_sh.at[pl.ds(base_n + NODE_BASE, CH)],
                        out_hbm.at[c].at[pl.ds(base_n + NODE_BASE, CH)])


def _make_agg():
    def body(feat_hbm, src_hbm, dst_hbm, out_hbm, src_v, dst_v,
             rows_a, rows_b, acc_sh, sem_ga, sem_gb, sem_sa, sem_sb):
        c = lax.axis_index("c")
        s = lax.axis_index("s")
        base_n, _ = _own_range(s)

        _fill(rows_a, W_EDGES, 0.0)
        _zero_acc(acc_sh, rows_a.at[pl.ds(0, CHZ)], rows_a.at[pl.ds(0, CH)],
                  base_n, s)
        plsc.subcore_barrier()

        row_base = (c * NS + s) * ROWS_PER_W
        bufs = (rows_a, rows_b)
        gsems = (sem_ga, sem_gb)
        ssems = (sem_sa, sem_sb)

        @pl.loop(0, ROWS_PER_W, step=IB)
        def _(b):
            pltpu.sync_copy(src_hbm.at[pl.ds(row_base + b, IB)], src_v)
            pltpu.sync_copy(dst_hbm.at[pl.ds(row_base + b, IB)], dst_v)

            gd = [None] * IB
            gd[0] = pltpu.async_copy(
                feat_hbm.at[src_v.at[0]], bufs[0], gsems[0])
            for j in range(IB):
                if j + 1 < IB:
                    gd[j + 1] = pltpu.async_copy(
                        feat_hbm.at[src_v.at[j + 1]],
                        bufs[(j + 1) % 2], gsems[(j + 1) % 2])
                gd[j].wait()
                pltpu.sync_copy(bufs[j % 2], acc_sh.at[dst_v.at[j]], add=True)

        plsc.subcore_barrier()
        _copy_out(acc_sh, out_hbm, c, base_n, s)

    return pl.kernel(
        body,
        out_type=jax.ShapeDtypeStruct((NC, N_NODES, D), jnp.float32),
        mesh=_sc_mesh(),
        scratch_types=[
            pltpu.VMEM((IB, W_EDGES), jnp.int32),
            pltpu.VMEM((IB, W_EDGES), jnp.int32),
            pltpu.VMEM((W_EDGES, D), jnp.float32),
            pltpu.VMEM((W_EDGES, D), jnp.float32),
            pltpu.VMEM_SHARED((N_NODES, D), jnp.float32),
            pltpu.SemaphoreType.DMA,
            pltpu.SemaphoreType.DMA,
            pltpu.SemaphoreType.DMA,
            pltpu.SemaphoreType.DMA,
        ])


def _make_deg():
    def body(dst_hbm, out_hbm, dst_v, ones_v, zbuf, acc_sh, dsem):
        c = lax.axis_index("c")
        s = lax.axis_index("s")
        base_n, _ = _own_range(s)

        _fill(zbuf, CHZ, 0.0)
        _fill(ones_v, W_EDGES, 1.0)
        _zero_acc(acc_sh, zbuf, zbuf.at[pl.ds(0, CH)], base_n, s)
        plsc.subcore_barrier()

        row_base = (c * NS + s) * ROWS_PER_W

        @pl.loop(0, ROWS_PER_W, step=IB)
        def _(b):
            pltpu.sync_copy(dst_hbm.at[pl.ds(row_base + b, IB)], dst_v)

            @pl.loop(0, IB)
            def _(j):
                pltpu.sync_copy(ones_v, acc_sh.at[dst_v.at[j]], add=True)

        plsc.subcore_barrier()
        _copy_out(acc_sh, out_hbm, c, base_n, s)

    return pl.kernel(
        body,
        out_type=jax.ShapeDtypeStruct((NC, N_NODES, D), jnp.float32),
        mesh=_sc_mesh(),
        scratch_types=[
            pltpu.VMEM((IB, W_EDGES), jnp.int32),
            pltpu.VMEM((W_EDGES, D), jnp.float32),
            pltpu.VMEM((CHZ, D), jnp.float32),
            pltpu.VMEM_SHARED((N_NODES, D), jnp.float32),
            pltpu.SemaphoreType.DMA,
        ])


_agg = functools.cache(_make_agg)
_deg = functools.cache(_make_deg)

BLK = 2000


def _mlp_block(x_ref, p_ref, dg_ref, wa_ref, ba_ref, wb_ref, bb_ref, o_ref):
    deg = dg_ref[0, :, 0:1] + dg_ref[1, :, 0:1]
    inv = 1.0 / jnp.maximum(deg, 1.0)
    h = x_ref[...] + (p_ref[0] + p_ref[1]) * inv
    dn = (((1,), (1,)), ((), ()))
    t = lax.dot_general(h, wa_ref[...], dn,
                        preferred_element_type=jnp.float32) + ba_ref[...]
    t = lax.dot_general(t, wb_ref[...], dn,
                        preferred_element_type=jnp.float32) + bb_ref[...]
    o_ref[...] = jnp.maximum(t, 0.0)


def _mlp(x, p, degp, wa_t, ba, wb_t, bb):
    return pl.pallas_call(
        _mlp_block,
        grid=(N_NODES // BLK,),
        in_specs=[
            pl.BlockSpec((BLK, D), lambda i: (i, 0)),
            pl.BlockSpec((NC, BLK, D), lambda i: (0, i, 0)),
            pl.BlockSpec((NC, BLK, D), lambda i: (0, i, 0)),
            pl.BlockSpec((D, D), lambda i: (0, 0)),
            pl.BlockSpec((1, D), lambda i: (0, 0)),
            pl.BlockSpec((D, D), lambda i: (0, 0)),
            pl.BlockSpec((1, D), lambda i: (0, 0)),
        ],
        out_specs=pl.BlockSpec((BLK, D), lambda i: (i, 0)),
        out_shape=jax.ShapeDtypeStruct((N_NODES, D), jnp.float32),
    )(x, p, degp, wa_t, ba.reshape(1, D), wb_t, bb.reshape(1, D))


def kernel(features, edge_index, W1a, b1a, W1b, b1b, W2a, b2a, W2b, b2b):
    src = edge_index[0].astype(jnp.int32).reshape(ROWS, W_EDGES)
    dst = edge_index[1].astype(jnp.int32).reshape(ROWS, W_EDGES)
    degp = _deg()(dst)
    p1 = _agg()(features, src, dst)
    x1 = _mlp(features, p1, degp, W1a, b1a, W1b, b1b)
    p2 = _agg()(x1, src, dst)
    out = _mlp(x1, p2, degp, W2a, b2a, W2b, b2b)
    return out


# speedup vs baseline: 1.3880x; 1.0017x over previous
import functools

import jax
import jax.numpy as jnp
from jax import lax
from jax.experimental import pallas as pl
from jax.experimental.pallas import tpu as pltpu
from jax.experimental.pallas import tpu_sc as plsc

N_NODES = 10000
D = 128
E = 320000
W_EDGES = 125
ROWS = E // W_EDGES
NC, NS = 2, 16
ROWS_PER_W = ROWS // (NC * NS)
NODE_BASE = 624
CH = 16
IB = 40


def _sc_mesh():
    return plsc.VectorSubcoreMesh(core_axis_name="c", subcore_axis_name="s",
                                  num_cores=NC, num_subcores=NS)


def _fill(ref, rows, value):
    @pl.loop(0, rows)
    def _(r):
        @pl.loop(0, D, step=16)
        def _(k):
            ref[r, pl.ds(k, 16)] = jnp.full((16,), value, jnp.float32)


def _own_range(s):
    base_n = s * NODE_BASE
    n_own = jnp.where(s == NS - 1, N_NODES - (NS - 1) * NODE_BASE, NODE_BASE)
    return base_n, n_own


CHZ = 104
CHO = 208


def _zero_acc(acc_sh, zsrc_big, zsrc_rem, base_n, s):
    @pl.loop(0, NODE_BASE, step=CHZ)
    def _(r):
        pltpu.sync_copy(zsrc_big, acc_sh.at[pl.ds(base_n + r, CHZ)])

    @pl.when(s == NS - 1)
    def _():
        pltpu.sync_copy(zsrc_rem, acc_sh.at[pl.ds(base_n + NODE_BASE, CH)])


def _copy_out(acc_sh, out_hbm, c, base_n, s):
    @pl.loop(0, NODE_BASE, step=CHO)
    def _(r):
        pltpu.sync_copy(acc_sh.at[pl.ds(base_n + r, CHO)],
                        out_hbm.at[c].at[pl.ds(base_n + r, CHO)])

    @pl.when(s == NS - 1)
    def _():
        pltpu.sync_copy(acc_sh.at[pl.ds(base_n + NODE_BASE, CH)],
                        out_hbm.at[c].at[pl.ds(base_n + NODE_BASE, CH)])


def _make_agg():
    def body(feat_hbm, src_hbm, dst_hbm, out_hbm, src_v, dst_v,
             rows_a, rows_b, acc_sh, sem_ga, sem_gb):
        c = lax.axis_index("c")
        s = lax.axis_index("s")
        base_n, _ = _own_range(s)

        _fill(rows_a, W_EDGES, 0.0)
        _zero_acc(acc_sh, rows_a.at[pl.ds(0, CHZ)], rows_a.at[pl.ds(0, CH)],
                  base_n, s)
        plsc.subcore_barrier()

        row_base = (c * NS + s) * ROWS_PER_W
        bufs = (rows_a, rows_b)
        gsems = (sem_ga, sem_gb)

        @pl.loop(0, ROWS_PER_W, step=IB)
        def _(b):
            pltpu.sync_copy(src_hbm.at[pl.ds(row_base + b, IB)], src_v)
            pltpu.sync_copy(dst_hbm.at[pl.ds(row_base + b, IB)], dst_v)

            gd = [None] * IB
            gd[0] = pltpu.async_copy(
                feat_hbm.at[src_v.at[0]], bufs[0], gsems[0])
            for j in range(IB):
                if j + 1 < IB:
                    gd[j + 1] = pltpu.async_copy(
                        feat_hbm.at[src_v.at[j + 1]],
                        bufs[(j + 1) % 2], gsems[(j + 1) % 2])
                gd[j].wait()
                pltpu.sync_copy(bufs[j % 2], acc_sh.at[dst_v.at[j]], add=True)

        plsc.subcore_barrier()
        _copy_out(acc_sh, out_hbm, c, base_n, s)

    return pl.kernel(
        body,
        out_type=jax.ShapeDtypeStruct((NC, N_NODES, D), jnp.float32),
        mesh=_sc_mesh(),
        scratch_types=[
            pltpu.VMEM((IB, W_EDGES), jnp.int32),
            pltpu.VMEM((IB, W_EDGES), jnp.int32),
            pltpu.VMEM((W_EDGES, D), jnp.float32),
            pltpu.VMEM((W_EDGES, D), jnp.float32),
            pltpu.VMEM_SHARED((N_NODES, D), jnp.float32),
            pltpu.SemaphoreType.DMA,
            pltpu.SemaphoreType.DMA,
        ])


def _make_deg():
    def body(dst_hbm, out_hbm, dst_v, ones_v, zbuf, acc_sh):
        c = lax.axis_index("c")
        s = lax.axis_index("s")
        base_n, _ = _own_range(s)

        _fill(zbuf, CHZ, 0.0)
        _fill(ones_v, W_EDGES, 1.0)
        _zero_acc(acc_sh, zbuf, zbuf.at[pl.ds(0, CH)], base_n, s)
        plsc.subcore_barrier()

        row_base = (c * NS + s) * ROWS_PER_W

        @pl.loop(0, ROWS_PER_W, step=IB)
        def _(b):
            pltpu.sync_copy(dst_hbm.at[pl.ds(row_base + b, IB)], dst_v)

            @pl.loop(0, IB)
            def _(j):
                pltpu.sync_copy(ones_v, acc_sh.at[dst_v.at[j]], add=True)

        plsc.subcore_barrier()
        _copy_out(acc_sh, out_hbm, c, base_n, s)

    return pl.kernel(
        body,
        out_type=jax.ShapeDtypeStruct((NC, N_NODES, D), jnp.float32),
        mesh=_sc_mesh(),
        scratch_types=[
            pltpu.VMEM((IB, W_EDGES), jnp.int32),
            pltpu.VMEM((W_EDGES, D), jnp.float32),
            pltpu.VMEM((CHZ, D), jnp.float32),
            pltpu.VMEM_SHARED((N_NODES, D), jnp.float32),
        ])


_agg = functools.cache(_make_agg)
_deg = functools.cache(_make_deg)

BLK = 2000


def _mlp_block(x_ref, p_ref, dg_ref, wa_ref, ba_ref, wb_ref, bb_ref, o_ref):
    deg = dg_ref[0, :, 0:1] + dg_ref[1, :, 0:1]
    inv = 1.0 / jnp.maximum(deg, 1.0)
    h = x_ref[...] + (p_ref[0] + p_ref[1]) * inv
    dn = (((1,), (1,)), ((), ()))
    t = lax.dot_general(h, wa_ref[...], dn,
                        preferred_element_type=jnp.float32) + ba_ref[...]
    t = lax.dot_general(t, wb_ref[...], dn,
                        preferred_element_type=jnp.float32) + bb_ref[...]
    o_ref[...] = jnp.maximum(t, 0.0)


def _mlp(x, p, degp, wa_t, ba, wb_t, bb):
    return pl.pallas_call(
        _mlp_block,
        grid=(N_NODES // BLK,),
        in_specs=[
            pl.BlockSpec((BLK, D), lambda i: (i, 0)),
            pl.BlockSpec((NC, BLK, D), lambda i: (0, i, 0)),
            pl.BlockSpec((NC, BLK, D), lambda i: (0, i, 0)),
            pl.BlockSpec((D, D), lambda i: (0, 0)),
            pl.BlockSpec((1, D), lambda i: (0, 0)),
            pl.BlockSpec((D, D), lambda i: (0, 0)),
            pl.BlockSpec((1, D), lambda i: (0, 0)),
        ],
        out_specs=pl.BlockSpec((BLK, D), lambda i: (i, 0)),
        out_shape=jax.ShapeDtypeStruct((N_NODES, D), jnp.float32),
    )(x, p, degp, wa_t, ba.reshape(1, D), wb_t, bb.reshape(1, D))


def kernel(features, edge_index, W1a, b1a, W1b, b1b, W2a, b2a, W2b, b2b):
    src = edge_index[0].astype(jnp.int32).reshape(ROWS, W_EDGES)
    dst = edge_index[1].astype(jnp.int32).reshape(ROWS, W_EDGES)
    degp = _deg()(dst)
    p1 = _agg()(features, src, dst)
    x1 = _mlp(features, p1, degp, W1a, b1a, W1b, b1b)
    p2 = _agg()(x1, src, dst)
    out = _mlp(x1, p2, degp, W2a, b2a, W2b, b2b)
    return out
